# baseline (speedup 1.0000x reference)
import jax
import jax.numpy as jnp
from jax import lax
from jax.experimental import pallas as pl
from jax.experimental.pallas import tpu as pltpu
from jax.experimental.pallas import tpu_sc as plsc

N_NODES = 50000
N_EDGES = 800000
EMB = 64
NUM_TYPES = 1000
NUM_ATTRS = 100000
MAX_DEPTH = 20
EDGE_IN = 16
EDGE_DIM = 16

NC, NS = 2, 16
NW = NC * NS
COLS_PER_W = EMB // NW
SEG = 3200
FULL_SEGS = N_NODES // SEG
TAIL = N_NODES - FULL_SEGS * SEG
NSEGS = FULL_SEGS + 1


def _gather_groups(n, cvec, tseg, aseg, dseg, oseg, acol, tcol, dtab):
    @plsc.parallel_loop(0, n // 64, 1, unroll=4)
    def group(i):
        for u in range(4):
            sl = pl.ds(i * 64 + u * 16, 16)
            d16 = jnp.minimum(dseg[sl], MAX_DEPTH)
            v = (plsc.load_gather(acol, [aseg[sl]])
                 + plsc.load_gather(tcol, [tseg[sl]])
                 + plsc.load_gather(dtab, [cvec, d16]))
            oseg[sl] = v
    base = (n // 64) * 64
    for u in range((n % 64) // 16):
        sl = pl.ds(base + u * 16, 16)
        d16 = jnp.minimum(dseg[sl], MAX_DEPTH)
        v = (plsc.load_gather(acol, [aseg[sl]])
             + plsc.load_gather(tcol, [tseg[sl]])
             + plsc.load_gather(dtab, [cvec, d16]))
        oseg[sl] = v


def _seg_len(s):
    return SEG if s < FULL_SEGS else TAIL


def _nodes_body(xT, dT, tT, aT, dthT, outT,
                acol, tcol, dtab,
                ts0, ts1, as0, as1, ds0, ds1, os0, os1,
                si0, si1, sw0, sw1):
    cc = lax.axis_index("c")
    ss = lax.axis_index("s")
    wid = ss * NC + cc
    pltpu.sync_copy(dthT, dtab)

    tbufs = (ts0, ts1)
    abufs = (as0, as1)
    dbufs = (ds0, ds1)
    obufs = (os0, os1)
    isems = (si0, si1)
    osems = (sw0, sw1)

    def issue_idx(s):
        b = s & 1
        n = _seg_len(s)
        off = s * SEG
        return [
            pltpu.async_copy(xT.at[0, pl.ds(off, n)],
                             tbufs[b].at[pl.ds(0, n)], isems[b]),
            pltpu.async_copy(xT.at[1, pl.ds(off, n)],
                             abufs[b].at[pl.ds(0, n)], isems[b]),
            pltpu.async_copy(dT.at[0, pl.ds(off, n)],
                             dbufs[b].at[pl.ds(0, n)], isems[b]),
        ]

    def col_body(q, carry):
        c = wid * COLS_PER_W + q
        pltpu.sync_copy(aT.at[c], acol)
        pltpu.sync_copy(tT.at[c], tcol)
        cvec = jnp.full((16,), c, dtype=jnp.int32)
        out_pending = [None, None]
        idx_pending = issue_idx(0)
        for s in range(NSEGS):
            b = s & 1
            n = _seg_len(s)
            for h in idx_pending:
                h.wait()
            if s + 1 < NSEGS:
                idx_pending = issue_idx(s + 1)
            if out_pending[b] is not None:
                out_pending[b].wait()
            _gather_groups(n, cvec, tbufs[b], abufs[b], dbufs[b], obufs[b],
                           acol, tcol, dtab)
            out_pending[b] = pltpu.async_copy(
                obufs[b].at[pl.ds(0, n)], outT.at[c, pl.ds(s * SEG, n)],
                osems[b])
        for b in (0, 1):
            if out_pending[b] is not None:
                out_pending[b].wait()
        return carry

    lax.fori_loop(0, COLS_PER_W, col_body, 0)


def _nodes_sc(xT, dT, tT, aT, dthT):
    mesh = plsc.VectorSubcoreMesh(core_axis_name="c", subcore_axis_name="s")
    return pl.kernel(
        _nodes_body,
        out_type=jax.ShapeDtypeStruct((EMB, N_NODES), jnp.float32),
        mesh=mesh,
        scratch_types=[
            pltpu.VMEM((NUM_ATTRS,), jnp.float32),
            pltpu.VMEM((NUM_TYPES,), jnp.float32),
            pltpu.VMEM((EMB, MAX_DEPTH + 1), jnp.float32),
            pltpu.VMEM((SEG,), jnp.int32),
            pltpu.VMEM((SEG,), jnp.int32),
            pltpu.VMEM((SEG,), jnp.int32),
            pltpu.VMEM((SEG,), jnp.int32),
            pltpu.VMEM((SEG,), jnp.int32),
            pltpu.VMEM((SEG,), jnp.int32),
            pltpu.VMEM((SEG,), jnp.float32),
            pltpu.VMEM((SEG,), jnp.float32),
            pltpu.SemaphoreType.DMA,
            pltpu.SemaphoreType.DMA,
            pltpu.SemaphoreType.DMA,
            pltpu.SemaphoreType.DMA,
        ],
        compiler_params=pltpu.CompilerParams(
            use_tc_tiling_on_sc=False, needs_layout_passes=False),
    )(xT, dT, tT, aT, dthT)


EDGE_BLK = 16000


def _edge_body(w_ref, x_ref, o_ref):
    o_ref[...] = lax.dot_general(
        w_ref[...], x_ref[...], (((0,), (0,)), ((), ())),
        preferred_element_type=jnp.float32)


def _edges_tc(eT, W_edge):
    return pl.pallas_call(
        _edge_body,
        grid=(N_EDGES // EDGE_BLK,),
        in_specs=[
            pl.BlockSpec((EDGE_IN, EDGE_DIM), lambda i: (0, 0)),
            pl.BlockSpec((EDGE_IN, EDGE_BLK), lambda i: (0, i)),
        ],
        out_specs=pl.BlockSpec((EDGE_DIM, EDGE_BLK), lambda i: (0, i)),
        out_shape=jax.ShapeDtypeStruct((EDGE_DIM, N_EDGES), jnp.float32),
    )(W_edge, eT)


def kernel(x, depth, edges, type_encoder, attribute_encoder, depth_encoder,
           W_edge):
    nodesT = _nodes_sc(x.T, depth.T, type_encoder.T, attribute_encoder.T,
                       depth_encoder.T)
    edges_outT = _edges_tc(edges.T, W_edge)
    return (nodesT.T, edges_outT.T)
